# baseline (speedup 1.0000x reference)
import functools

import jax
import jax.numpy as jnp
from jax import lax
from jax.experimental import pallas as pl
from jax.experimental.pallas import tpu as pltpu
from jax.experimental.pallas import tpu_sc as plsc

NC = 2
NS = 16
NW = NC * NS
LANE = 128
K = 4
NBUF = 3


def kernel(indices, embeddings):
    B, S = indices.shape
    V, D = embeddings.shape
    total = B * S
    per_w = total // NW
    grp = K * LANE
    n_grp = per_w // grp
    assert per_w * NW == total and n_grp * grp == per_w

    idx4 = indices.reshape(NW, n_grp, grp)
    mesh = plsc.VectorSubcoreMesh(core_axis_name="c", subcore_axis_name="s")

    @functools.partial(
        pl.kernel,
        out_type=jax.ShapeDtypeStruct((total, D), jnp.float32),
        mesh=mesh,
        compiler_params=pltpu.CompilerParams(use_tc_tiling_on_sc=False),
        scratch_types=[
            pltpu.VMEM((n_grp, grp), jnp.int32),
            pltpu.VMEM((NBUF, grp, D), jnp.float32),
            pltpu.SemaphoreType.DMA((NBUF,)),
            pltpu.SemaphoreType.DMA((NBUF,)),
        ],
    )
    def gather_kernel(idx_hbm, tab_hbm, out_hbm, idx_v, rows, gsem, wsem):
        wid = lax.axis_index("s") * NC + lax.axis_index("c")
        base = wid * per_w
        pltpu.sync_copy(idx_hbm.at[wid], idx_v)

        def fire_gather(g, s):
            pltpu.async_copy(tab_hbm.at[idx_v.at[g]], rows.at[s], gsem.at[s])

        def wait_gather(g, s):
            pltpu.make_async_copy(
                tab_hbm.at[idx_v.at[g]], rows.at[s], gsem.at[s]).wait()

        def fire_write(g, s):
            pltpu.async_copy(
                rows.at[s], out_hbm.at[pl.ds(base + g * grp, grp)], wsem.at[s])

        def wait_write(g, s):
            pltpu.make_async_copy(
                rows.at[s], out_hbm.at[pl.ds(base + g * grp, grp)],
                wsem.at[s]).wait()

        for b in range(NBUF):
            fire_gather(b, b)

        def body(g, carry):
            s = lax.rem(g, NBUF)
            wait_gather(g, s)
            fire_write(g, s)
            g2 = g + NBUF

            @pl.when(g2 < n_grp)
            def _():
                wait_write(g, s)
                fire_gather(g2, s)

            return carry

        lax.fori_loop(0, n_grp, body, 0)

        for b in range(NBUF):
            g = n_grp - NBUF + b
            wait_write(g, g % NBUF)

    out = gather_kernel(idx4, embeddings)
    return out.reshape(B, S, D)


# speedup vs baseline: 1.2248x; 1.2248x over previous
import functools

import jax
import jax.numpy as jnp
from jax import lax
from jax.experimental import pallas as pl
from jax.experimental.pallas import tpu as pltpu
from jax.experimental.pallas import tpu_sc as plsc

NC = 2
NS = 16
NW = NC * NS
BCHUNK = 1
NBUF = 3


def kernel(indices, embeddings):
    B, S = indices.shape
    V, D = embeddings.shape
    b_per_w = B // NW
    grp = BCHUNK * S
    n_grp = b_per_w // BCHUNK
    per_w = b_per_w * S
    assert b_per_w * NW == B and n_grp * BCHUNK == b_per_w

    tab128 = jnp.pad(embeddings, ((0, 0), (0, 128 - D)))
    idx2 = indices.reshape(NW, per_w)
    mesh = plsc.VectorSubcoreMesh(core_axis_name="c", subcore_axis_name="s")

    @functools.partial(
        pl.kernel,
        out_type=jax.ShapeDtypeStruct((B * S, 128), jnp.float32),
        mesh=mesh,
        compiler_params=pltpu.CompilerParams(use_tc_tiling_on_sc=True),
        scratch_types=[
            pltpu.VMEM((per_w,), jnp.int32),
            pltpu.VMEM((NBUF, grp, 128), jnp.float32),
            pltpu.SemaphoreType.DMA((NBUF,)),
            pltpu.SemaphoreType.DMA((NBUF,)),
        ],
    )
    def gather_kernel(idx_hbm, tab_hbm, out_hbm, idx_v, rows, gsem, wsem):
        wid = lax.axis_index("s") * NC + lax.axis_index("c")
        t0 = wid * per_w
        pltpu.sync_copy(idx_hbm.at[wid], idx_v)

        def fire_gather(g, s):
            pltpu.async_copy(
                tab_hbm.at[idx_v.at[pl.ds(g * grp, grp)]], rows.at[s],
                gsem.at[s])

        def wait_gather(g, s):
            pltpu.make_async_copy(
                tab_hbm.at[idx_v.at[pl.ds(g * grp, grp)]], rows.at[s],
                gsem.at[s]).wait()

        def fire_write(g, s):
            pltpu.async_copy(
                rows.at[s], out_hbm.at[pl.ds(t0 + g * grp, grp)], wsem.at[s])

        def wait_write(g, s):
            pltpu.make_async_copy(
                rows.at[s], out_hbm.at[pl.ds(t0 + g * grp, grp)],
                wsem.at[s]).wait()

        for b in range(NBUF):
            fire_gather(b, b)

        def body(g, carry):
            s = lax.rem(g, NBUF)
            wait_gather(g, s)
            fire_write(g, s)
            g2 = g + NBUF

            @pl.when(g2 < n_grp)
            def _():
                wait_write(g, s)
                fire_gather(g2, s)

            return carry

        lax.fori_loop(0, n_grp, body, 0)

        for b in range(NBUF):
            g = n_grp - NBUF + b
            wait_write(g, g % NBUF)

    out128 = gather_kernel(idx2, tab128)
    return out128[:, :D].reshape(B, S, D)
